# Initial kernel scaffold; baseline (speedup 1.0000x reference)
import jax
import jax.numpy as jnp
from jax.experimental import pallas as pl


def kernel(x, node_in, node_out, relation, edge_weight, W, b, rel_table):
    raise NotImplementedError("write your pallas kernel here")


# speedup vs baseline: 3.2982x; 3.2982x over previous
import functools

import jax
import jax.numpy as jnp
from jax import lax
from jax.experimental import pallas as pl
from jax.experimental.pallas import tpu as pltpu
from jax.experimental.pallas import tpu_sc as plsc

N_NODES = 10000
N_PAD = 10240
D = 128
N_REL = 64
E = 320000

NC = 2
NS = 16
NW = NC * NS
CH = 128
NCHUNKS = E // CH
TRIPS = (NCHUNKS + NW - 1) // NW
RPS = N_PAD // NS


def _mm_body(x_ref, wt_ref, y_ref):
    y_ref[...] = jnp.dot(x_ref[...], wt_ref[...],
                         preferred_element_type=jnp.float32)


def _node_linear(x, wt):
    blk = 2000
    return pl.pallas_call(
        _mm_body,
        grid=(N_NODES // blk,),
        in_specs=[
            pl.BlockSpec((blk, D), lambda i: (i, 0)),
            pl.BlockSpec((D, D), lambda i: (0, 0)),
        ],
        out_specs=pl.BlockSpec((blk, D), lambda i: (i, 0)),
        out_shape=jax.ShapeDtypeStruct((N_NODES, D), jnp.float32),
    )(x, wt)


def _sc_body(y_hbm, nin_hbm, nout_hbm, rel_hbm, ew_hbm, rtab_hbm, za_hbm,
             zd_hbm,
             acc_out, deg_out,
             acc_sh, deg_sh, rows_v, rtab_v, ii_v, oo_v, rr_v, w_v, sem):
    cid = lax.axis_index("c")
    sid = lax.axis_index("s")
    wid = sid * NC + cid

    pltpu.sync_copy(rtab_hbm, rtab_v)
    pltpu.sync_copy(za_hbm, acc_sh.at[pl.ds(sid * RPS, RPS)])
    pltpu.sync_copy(zd_hbm, deg_sh.at[pl.ds(sid * RPS, RPS)])
    plsc.subcore_barrier()

    lanes = lax.iota(jnp.int32, 16)

    def trip(t, carry):
        c = wid + t * NW

        @pl.when(c < NCHUNKS)
        def _():
            base = c * CH
            pltpu.sync_copy(nin_hbm.at[pl.ds(base, CH)], ii_v)
            pltpu.sync_copy(ew_hbm.at[pl.ds(base, CH)], w_v)
            pltpu.sync_copy(nout_hbm.at[pl.ds(base, CH)], oo_v.at[0])
            pltpu.sync_copy(rel_hbm.at[pl.ds(base, CH)], rr_v)
            pltpu.async_copy(y_hbm.at[ii_v], rows_v, sem).wait()

            pltpu.sync_copy(w_v, deg_sh.at[oo_v.at[0]], add=True)

            def ebody(e, carry2):
                esplat = jnp.full((16,), e, jnp.int32)
                wb = plsc.load_gather(w_v, [esplat])
                rb = plsc.load_gather(rr_v, [esplat])
                for j in range(D // 16):
                    sl = (e, pl.ds(j * 16, 16))
                    rel = plsc.load_gather(rtab_v, [rb, lanes + (j * 16)])
                    rows_v[sl] = (rows_v[sl] + rel) * wb
                return carry2

            lax.fori_loop(0, CH, ebody, 0)

            pltpu.sync_copy(rows_v, acc_sh.at[oo_v.at[0]], add=True)

        return carry

    lax.fori_loop(0, TRIPS, trip, 0)
    plsc.subcore_barrier()

    pltpu.sync_copy(acc_sh.at[pl.ds(sid * RPS, RPS)],
                    acc_out.at[cid, 0, pl.ds(sid * RPS, RPS)])
    pltpu.sync_copy(deg_sh.at[pl.ds(sid * RPS, RPS)],
                    deg_out.at[cid, 0, pl.ds(sid * RPS, RPS)])


@functools.cache
def _sc_agg():
    mesh = plsc.VectorSubcoreMesh(core_axis_name="c", subcore_axis_name="s")
    return pl.kernel(
        _sc_body,
        out_type=(
            jax.ShapeDtypeStruct((NC, 1, N_PAD, D), jnp.float32),
            jax.ShapeDtypeStruct((NC, 1, N_PAD), jnp.float32),
        ),
        mesh=mesh,
        compiler_params=pltpu.CompilerParams(needs_layout_passes=False),
        scratch_types=[
            pltpu.VMEM_SHARED((N_PAD, D), jnp.float32),
            pltpu.VMEM_SHARED((N_PAD,), jnp.float32),
            pltpu.VMEM((CH, D), jnp.float32),
            pltpu.VMEM((N_REL, D), jnp.float32),
            pltpu.VMEM((CH,), jnp.int32),
            pltpu.VMEM((1, CH), jnp.int32),
            pltpu.VMEM((CH,), jnp.int32),
            pltpu.VMEM((CH,), jnp.float32),
            pltpu.SemaphoreType.DMA,
        ],
    )


def _fin_body(acc_ref, deg_ref, b_ref, o_ref):
    acc = acc_ref[0] + acc_ref[1]
    deg = deg_ref[0] + deg_ref[1]
    num = acc + deg * b_ref[...]
    o_ref[...] = jnp.maximum(num / (deg + 1e-6), 0.0)


def _finalize(acc_p, deg_p, b2):
    blk = 2000
    return pl.pallas_call(
        _fin_body,
        grid=(N_NODES // blk,),
        in_specs=[
            pl.BlockSpec((NC, blk, D), lambda i: (0, i, 0)),
            pl.BlockSpec((NC, blk, 1), lambda i: (0, i, 0)),
            pl.BlockSpec((1, D), lambda i: (0, 0)),
        ],
        out_specs=pl.BlockSpec((blk, D), lambda i: (i, 0)),
        out_shape=jax.ShapeDtypeStruct((N_NODES, D), jnp.float32),
    )(acc_p, deg_p, b2)


def kernel(x, node_in, node_out, relation, edge_weight, W, b, rel_table):
    y = _node_linear(x, W.T)
    za = jnp.zeros((RPS, D), jnp.float32)
    zd = jnp.zeros((RPS,), jnp.float32)
    acc_p, deg_p = _sc_agg()(
        y,
        node_in.astype(jnp.int32),
        node_out.astype(jnp.int32),
        relation.astype(jnp.int32),
        edge_weight,
        rel_table,
        za,
        zd,
    )
    acc_p = acc_p.reshape(NC, N_PAD, D)[:, :N_NODES]
    deg_p = deg_p.reshape(NC, N_PAD, 1)[:, :N_NODES]
    return _finalize(acc_p, deg_p, b.reshape(1, D))


# speedup vs baseline: 4.1051x; 1.2446x over previous
import functools

import jax
import jax.numpy as jnp
from jax import lax
from jax.experimental import pallas as pl
from jax.experimental.pallas import tpu as pltpu
from jax.experimental.pallas import tpu_sc as plsc

N_NODES = 10000
N_PAD = 10240
D = 128
N_REL = 64
E = 320000

NC = 2
NS = 16
NW = NC * NS
CH = 128
NCHUNKS = E // CH
PAIRS = (NCHUNKS // NW + 2) // 2
RPS = N_PAD // NS


def _mm_body(x_ref, wt_ref, y_ref):
    y_ref[...] = jnp.dot(x_ref[...], wt_ref[...],
                         preferred_element_type=jnp.float32)


def _node_linear(x, wt):
    blk = 2000
    return pl.pallas_call(
        _mm_body,
        grid=(N_NODES // blk,),
        in_specs=[
            pl.BlockSpec((blk, D), lambda i: (i, 0)),
            pl.BlockSpec((D, D), lambda i: (0, 0)),
        ],
        out_specs=pl.BlockSpec((blk, D), lambda i: (i, 0)),
        out_shape=jax.ShapeDtypeStruct((N_NODES, D), jnp.float32),
    )(x, wt)


def _sc_body(y_hbm, ed_hbm, ew_hbm, rtab_hbm, za_hbm, zd_hbm,
             acc_out, deg_out,
             acc_sh, deg_sh, rows0, rows1, ib0, ib1, wv0, wv1, rtab_v,
             sem_g0, sem_g1, sem_i0, sem_i1):
    cid = lax.axis_index("c")
    sid = lax.axis_index("s")
    wid = sid * NC + cid

    pltpu.sync_copy(rtab_hbm, rtab_v)
    pltpu.sync_copy(za_hbm, acc_sh.at[pl.ds(sid * RPS, RPS)])
    pltpu.sync_copy(zd_hbm, deg_sh.at[pl.ds(sid * RPS, RPS)])
    plsc.subcore_barrier()

    lanes = lax.iota(jnp.int32, 16)
    zeros16 = jnp.zeros((16,), jnp.int32)
    two16 = jnp.full((16,), 2, jnp.int32)

    def idx_start(c, ib, wv, sem):
        pltpu.async_copy(ed_hbm.at[c], ib, sem)
        pltpu.async_copy(ew_hbm.at[c], wv, sem)

    def idx_wait(c, ib, wv, sem):
        pltpu.make_async_copy(ed_hbm.at[c], ib, sem).wait()
        pltpu.make_async_copy(ew_hbm.at[c], wv, sem).wait()

    def g_start(ib, rows, sem):
        pltpu.async_copy(y_hbm.at[ib.at[0]], rows, sem)

    def g_wait(ib, rows, sem):
        pltpu.make_async_copy(y_hbm.at[ib.at[0]], rows, sem).wait()

    idx_start(wid, ib0, wv0, sem_i0)
    idx_start(wid + NW, ib1, wv1, sem_i1)
    idx_wait(wid, ib0, wv0, sem_i0)
    g_start(ib0, rows0, sem_g0)

    bufs = ((rows0, ib0, wv0, sem_g0, sem_i0),
            (rows1, ib1, wv1, sem_g1, sem_i1))

    def pair(tt, carry):
        for db in range(2):
            t = tt * 2 + db
            rows_b, ib_b, wv_b, sem_gb, sem_ib = bufs[db]
            rows_n, ib_n, wv_n, sem_gn, sem_in = bufs[1 - db]
            c = wid + t * NW

            @pl.when(c + NW < NCHUNKS)
            def _():
                idx_wait(c + NW, ib_n, wv_n, sem_in)
                g_start(ib_n, rows_n, sem_gn)

            @pl.when(c < NCHUNKS)
            def _():
                g_wait(ib_b, rows_b, sem_gb)

                def ebody(e, carry2):
                    esplat = jnp.full((16,), e, jnp.int32)
                    wb = plsc.load_gather(wv_b, [zeros16, esplat])
                    rb = plsc.load_gather(ib_b, [two16, esplat])
                    for j in range(D // 16):
                        sl = (e, pl.ds(j * 16, 16))
                        rel = plsc.load_gather(rtab_v, [rb, lanes + (j * 16)])
                        rows_b[sl] = (rows_b[sl] + rel) * wb
                    return carry2

                lax.fori_loop(0, CH, ebody, 0)

                pltpu.sync_copy(wv_b.at[0], deg_sh.at[ib_b.at[1]], add=True)
                pltpu.sync_copy(rows_b, acc_sh.at[ib_b.at[1]], add=True)

            @pl.when(c + 2 * NW < NCHUNKS)
            def _():
                idx_start(c + 2 * NW, ib_b, wv_b, sem_ib)

        return carry

    lax.fori_loop(0, PAIRS, pair, 0)
    plsc.subcore_barrier()

    pltpu.sync_copy(acc_sh.at[pl.ds(sid * RPS, RPS)],
                    acc_out.at[cid, 0, pl.ds(sid * RPS, RPS)])
    pltpu.sync_copy(deg_sh.at[pl.ds(sid * RPS, RPS)],
                    deg_out.at[cid, 0, pl.ds(sid * RPS, RPS)])


@functools.cache
def _sc_agg():
    mesh = plsc.VectorSubcoreMesh(core_axis_name="c", subcore_axis_name="s")
    return pl.kernel(
        _sc_body,
        out_type=(
            jax.ShapeDtypeStruct((NC, 1, N_PAD, D), jnp.float32),
            jax.ShapeDtypeStruct((NC, 1, N_PAD), jnp.float32),
        ),
        mesh=mesh,
        compiler_params=pltpu.CompilerParams(needs_layout_passes=False),
        scratch_types=[
            pltpu.VMEM_SHARED((N_PAD, D), jnp.float32),
            pltpu.VMEM_SHARED((N_PAD,), jnp.float32),
            pltpu.VMEM((CH, D), jnp.float32),
            pltpu.VMEM((CH, D), jnp.float32),
            pltpu.VMEM((3, CH), jnp.int32),
            pltpu.VMEM((3, CH), jnp.int32),
            pltpu.VMEM((1, CH), jnp.float32),
            pltpu.VMEM((1, CH), jnp.float32),
            pltpu.VMEM((N_REL, D), jnp.float32),
            pltpu.SemaphoreType.DMA,
            pltpu.SemaphoreType.DMA,
            pltpu.SemaphoreType.DMA,
            pltpu.SemaphoreType.DMA,
        ],
    )


def _fin_body(acc_ref, deg_ref, b_ref, o_ref):
    acc = acc_ref[0] + acc_ref[1]
    deg = deg_ref[0] + deg_ref[1]
    num = acc + deg * b_ref[...]
    o_ref[...] = jnp.maximum(num / (deg + 1e-6), 0.0)


def _finalize(acc_p, deg_p, b2):
    blk = 2000
    return pl.pallas_call(
        _fin_body,
        grid=(N_NODES // blk,),
        in_specs=[
            pl.BlockSpec((NC, blk, D), lambda i: (0, i, 0)),
            pl.BlockSpec((NC, blk, 1), lambda i: (0, i, 0)),
            pl.BlockSpec((1, D), lambda i: (0, 0)),
        ],
        out_specs=pl.BlockSpec((blk, D), lambda i: (i, 0)),
        out_shape=jax.ShapeDtypeStruct((N_NODES, D), jnp.float32),
    )(acc_p, deg_p, b2)


def kernel(x, node_in, node_out, relation, edge_weight, W, b, rel_table):
    y = _node_linear(x, W.T)
    nin = node_in.astype(jnp.int32).reshape(NCHUNKS, CH)
    nout = node_out.astype(jnp.int32).reshape(NCHUNKS, CH)
    rel = relation.astype(jnp.int32).reshape(NCHUNKS, CH)
    ed = jnp.stack([nin, nout, rel], axis=1)
    ew = edge_weight.reshape(NCHUNKS, 1, CH)
    za = jnp.zeros((RPS, D), jnp.float32)
    zd = jnp.zeros((RPS,), jnp.float32)
    acc_p, deg_p = _sc_agg()(y, ed, ew, rel_table, za, zd)
    acc_p = acc_p.reshape(NC, N_PAD, D)[:, :N_NODES]
    deg_p = deg_p.reshape(NC, N_PAD, 1)[:, :N_NODES]
    return _finalize(acc_p, deg_p, b.reshape(1, D))


# speedup vs baseline: 10.5482x; 2.5696x over previous
import functools

import jax
import jax.numpy as jnp
from jax import lax
from jax.experimental import pallas as pl
from jax.experimental.pallas import tpu as pltpu
from jax.experimental.pallas import tpu_sc as plsc

N_NODES = 10000
N_PAD = 10240
D = 128
N_REL = 64
E = 320000

NC = 2
NS = 16
NW = NC * NS
CH = 128
NCHUNKS = E // CH
PAIRS = (NCHUNKS // NW + 2) // 2
RPS = N_PAD // NS


def _mm_body(x_ref, wt_ref, y_ref):
    y_ref[...] = jnp.dot(x_ref[...], wt_ref[...],
                         preferred_element_type=jnp.float32)


def _node_linear(x, wt):
    blk = 2000
    return pl.pallas_call(
        _mm_body,
        grid=(N_NODES // blk,),
        in_specs=[
            pl.BlockSpec((blk, D), lambda i: (i, 0)),
            pl.BlockSpec((D, D), lambda i: (0, 0)),
        ],
        out_specs=pl.BlockSpec((blk, D), lambda i: (i, 0)),
        out_shape=jax.ShapeDtypeStruct((N_NODES, D), jnp.float32),
    )(x, wt)


def _sc_body(y_hbm, ed_hbm, ew_hbm, rtab_hbm, za_hbm, zd_hbm,
             acc_out, deg_out,
             acc_sh, deg_sh, rows0, rows1, ib0, ib1, wv0, wv1, rtab_v,
             sem_g0, sem_g1, sem_i0, sem_i1):
    cid = lax.axis_index("c")
    sid = lax.axis_index("s")
    wid = sid * NC + cid

    pltpu.sync_copy(rtab_hbm, rtab_v)
    pltpu.sync_copy(za_hbm, acc_sh.at[pl.ds(sid * RPS, RPS)])
    pltpu.sync_copy(zd_hbm, deg_sh.at[pl.ds(sid * RPS, RPS)])
    plsc.subcore_barrier()

    lanes = lax.iota(jnp.int32, 16)
    zeros16 = jnp.zeros((16,), jnp.int32)
    two16 = jnp.full((16,), 2, jnp.int32)

    def idx_start(c, ib, wv, sem):
        pltpu.async_copy(ed_hbm.at[c], ib, sem)
        pltpu.async_copy(ew_hbm.at[c], wv, sem)

    def idx_wait(c, ib, wv, sem):
        pltpu.make_async_copy(ed_hbm.at[c], ib, sem).wait()
        pltpu.make_async_copy(ew_hbm.at[c], wv, sem).wait()

    def g_start(ib, rows, sem):
        pltpu.async_copy(y_hbm.at[ib.at[0]], rows, sem)

    def g_wait(ib, rows, sem):
        pltpu.make_async_copy(y_hbm.at[ib.at[0]], rows, sem).wait()

    idx_start(wid, ib0, wv0, sem_i0)
    idx_start(wid + NW, ib1, wv1, sem_i1)
    idx_wait(wid, ib0, wv0, sem_i0)
    g_start(ib0, rows0, sem_g0)

    bufs = ((rows0, ib0, wv0, sem_g0, sem_i0),
            (rows1, ib1, wv1, sem_g1, sem_i1))

    def pair(tt, carry):
        for db in range(2):
            t = tt * 2 + db
            rows_b, ib_b, wv_b, sem_gb, sem_ib = bufs[db]
            rows_n, ib_n, wv_n, sem_gn, sem_in = bufs[1 - db]
            c = wid + t * NW

            @pl.when(c + NW < NCHUNKS)
            def _():
                idx_wait(c + NW, ib_n, wv_n, sem_in)
                g_start(ib_n, rows_n, sem_gn)

            @pl.when(c < NCHUNKS)
            def _():
                g_wait(ib_b, rows_b, sem_gb)

                @plsc.parallel_loop(0, CH, step=1, unroll=8)
                def _(e):
                    esplat = jnp.full((16,), e, jnp.int32)
                    wb = plsc.load_gather(wv_b, [zeros16, esplat])
                    rb = plsc.load_gather(ib_b, [two16, esplat])
                    for j in range(D // 16):
                        sl = (e, pl.ds(j * 16, 16))
                        rel = plsc.load_gather(rtab_v, [rb, lanes + (j * 16)])
                        rows_b[sl] = (rows_b[sl] + rel) * wb

                pltpu.sync_copy(wv_b.at[0], deg_sh.at[ib_b.at[1]], add=True)
                pltpu.sync_copy(rows_b, acc_sh.at[ib_b.at[1]], add=True)

            @pl.when(c + 2 * NW < NCHUNKS)
            def _():
                idx_start(c + 2 * NW, ib_b, wv_b, sem_ib)

        return carry

    lax.fori_loop(0, PAIRS, pair, 0)
    plsc.subcore_barrier()

    pltpu.sync_copy(acc_sh.at[pl.ds(sid * RPS, RPS)],
                    acc_out.at[cid, 0, pl.ds(sid * RPS, RPS)])
    pltpu.sync_copy(deg_sh.at[pl.ds(sid * RPS, RPS)],
                    deg_out.at[cid, 0, pl.ds(sid * RPS, RPS)])


@functools.cache
def _sc_agg():
    mesh = plsc.VectorSubcoreMesh(core_axis_name="c", subcore_axis_name="s")
    return pl.kernel(
        _sc_body,
        out_type=(
            jax.ShapeDtypeStruct((NC, 1, N_PAD, D), jnp.float32),
            jax.ShapeDtypeStruct((NC, 1, N_PAD), jnp.float32),
        ),
        mesh=mesh,
        compiler_params=pltpu.CompilerParams(needs_layout_passes=False),
        scratch_types=[
            pltpu.VMEM_SHARED((N_PAD, D), jnp.float32),
            pltpu.VMEM_SHARED((N_PAD,), jnp.float32),
            pltpu.VMEM((CH, D), jnp.float32),
            pltpu.VMEM((CH, D), jnp.float32),
            pltpu.VMEM((3, CH), jnp.int32),
            pltpu.VMEM((3, CH), jnp.int32),
            pltpu.VMEM((1, CH), jnp.float32),
            pltpu.VMEM((1, CH), jnp.float32),
            pltpu.VMEM((N_REL, D), jnp.float32),
            pltpu.SemaphoreType.DMA,
            pltpu.SemaphoreType.DMA,
            pltpu.SemaphoreType.DMA,
            pltpu.SemaphoreType.DMA,
        ],
    )


def _fin_body(acc_ref, deg_ref, b_ref, o_ref):
    acc = acc_ref[0] + acc_ref[1]
    deg = deg_ref[0] + deg_ref[1]
    num = acc + deg * b_ref[...]
    o_ref[...] = jnp.maximum(num / (deg + 1e-6), 0.0)


def _finalize(acc_p, deg_p, b2):
    blk = 2000
    return pl.pallas_call(
        _fin_body,
        grid=(N_NODES // blk,),
        in_specs=[
            pl.BlockSpec((NC, blk, D), lambda i: (0, i, 0)),
            pl.BlockSpec((NC, blk, 1), lambda i: (0, i, 0)),
            pl.BlockSpec((1, D), lambda i: (0, 0)),
        ],
        out_specs=pl.BlockSpec((blk, D), lambda i: (i, 0)),
        out_shape=jax.ShapeDtypeStruct((N_NODES, D), jnp.float32),
    )(acc_p, deg_p, b2)


def kernel(x, node_in, node_out, relation, edge_weight, W, b, rel_table):
    y = _node_linear(x, W.T)
    nin = node_in.astype(jnp.int32).reshape(NCHUNKS, CH)
    nout = node_out.astype(jnp.int32).reshape(NCHUNKS, CH)
    rel = relation.astype(jnp.int32).reshape(NCHUNKS, CH)
    ed = jnp.stack([nin, nout, rel], axis=1)
    ew = edge_weight.reshape(NCHUNKS, 1, CH)
    za = jnp.zeros((RPS, D), jnp.float32)
    zd = jnp.zeros((RPS,), jnp.float32)
    acc_p, deg_p = _sc_agg()(y, ed, ew, rel_table, za, zd)
    acc_p = acc_p.reshape(NC, N_PAD, D)[:, :N_NODES]
    deg_p = deg_p.reshape(NC, N_PAD, 1)[:, :N_NODES]
    return _finalize(acc_p, deg_p, b.reshape(1, D))


# speedup vs baseline: 12.8301x; 1.2163x over previous
import functools

import jax
import jax.numpy as jnp
from jax import lax
from jax.experimental import pallas as pl
from jax.experimental.pallas import tpu as pltpu
from jax.experimental.pallas import tpu_sc as plsc

N_NODES = 10000
N_PAD = 10240
D = 128
N_REL = 64
E = 320000

NC = 2
NS = 16
NW = NC * NS
CH = 128
NCHUNKS = E // CH
PAIRS = (NCHUNKS // NW + 2) // 2
RPS = N_PAD // NS


def _mm_body(x_ref, wt_ref, y_ref):
    y_ref[...] = jnp.dot(x_ref[...], wt_ref[...],
                         preferred_element_type=jnp.float32)


def _node_linear(x, wt):
    blk = 2000
    return pl.pallas_call(
        _mm_body,
        grid=(N_NODES // blk,),
        in_specs=[
            pl.BlockSpec((blk, D), lambda i: (i, 0)),
            pl.BlockSpec((D, D), lambda i: (0, 0)),
        ],
        out_specs=pl.BlockSpec((blk, D), lambda i: (i, 0)),
        out_shape=jax.ShapeDtypeStruct((N_NODES, D), jnp.float32),
    )(x, wt)


def _sc_body(y_hbm, ed_hbm, ew_hbm, rtab_hbm, za_hbm, zd_hbm,
             acc_out, deg_out,
             acc_sh, deg_sh, rows0, rows1, ib0, ib1, wv0, wv1,
             so0, so1, sw0, sw1, rtab_v,
             sem_g0, sem_g1, sem_i0, sem_i1, sem_s0, sem_s1):
    cid = lax.axis_index("c")
    sid = lax.axis_index("s")
    wid = sid * NC + cid

    pltpu.sync_copy(rtab_hbm, rtab_v)
    pltpu.sync_copy(za_hbm, acc_sh.at[pl.ds(sid * RPS, RPS)])
    pltpu.sync_copy(zd_hbm, deg_sh.at[pl.ds(sid * RPS, RPS)])
    plsc.subcore_barrier()

    lanes = lax.iota(jnp.int32, 16)
    zeros16 = jnp.zeros((16,), jnp.int32)
    two16 = jnp.full((16,), 2, jnp.int32)

    def idx_start(c, ib, wv, sem):
        pltpu.async_copy(ed_hbm.at[c], ib, sem)
        pltpu.async_copy(ew_hbm.at[c], wv, sem)

    def idx_wait(c, ib, wv, sem):
        pltpu.make_async_copy(ed_hbm.at[c], ib, sem).wait()
        pltpu.make_async_copy(ew_hbm.at[c], wv, sem).wait()

    def g_start(ib, rows, sem):
        pltpu.async_copy(y_hbm.at[ib.at[0]], rows, sem)

    def g_wait(ib, rows, sem):
        pltpu.make_async_copy(y_hbm.at[ib.at[0]], rows, sem).wait()

    def s_start(rows, so, sw, sem):
        pltpu.async_copy(rows, acc_sh.at[so.at[0]], sem)
        pltpu.async_copy(sw.at[0], deg_sh.at[so.at[0]], sem)

    def s_wait(rows, so, sw, sem):
        pltpu.make_async_copy(rows, acc_sh.at[so.at[0]], sem).wait()
        pltpu.make_async_copy(sw.at[0], deg_sh.at[so.at[0]], sem).wait()

    idx_start(wid, ib0, wv0, sem_i0)
    idx_start(wid + NW, ib1, wv1, sem_i1)
    idx_wait(wid, ib0, wv0, sem_i0)
    g_start(ib0, rows0, sem_g0)

    bufs = ((rows0, ib0, wv0, so0, sw0, sem_g0, sem_i0, sem_s0),
            (rows1, ib1, wv1, so1, sw1, sem_g1, sem_i1, sem_s1))

    def pair(tt, carry):
        for db in range(2):
            t = tt * 2 + db
            rows_b, ib_b, wv_b, so_b, sw_b, sem_gb, sem_ib, sem_sb = bufs[db]
            rows_n, ib_n, wv_n, so_n, sw_n, sem_gn, sem_in, sem_sn = (
                bufs[1 - db])
            c = wid + t * NW

            @pl.when(c + NW < NCHUNKS)
            def _():
                idx_wait(c + NW, ib_n, wv_n, sem_in)

                @pl.when(c >= NW)
                def _():
                    s_wait(rows_n, so_n, sw_n, sem_sn)

                g_start(ib_n, rows_n, sem_gn)

            @pl.when(c < NCHUNKS)
            def _():
                g_wait(ib_b, rows_b, sem_gb)

                for g in range(CH // 16):
                    gs = pl.ds(g * 16, 16)
                    so_b[0, gs] = ib_b[1, gs]
                    sw_b[0, gs] = wv_b[0, gs]

                @plsc.parallel_loop(0, CH, step=1, unroll=8)
                def _(e):
                    esplat = jnp.full((16,), e, jnp.int32)
                    wb = plsc.load_gather(wv_b, [zeros16, esplat])
                    rb = plsc.load_gather(ib_b, [two16, esplat])
                    for j in range(D // 16):
                        sl = (e, pl.ds(j * 16, 16))
                        rel = plsc.load_gather(rtab_v, [rb, lanes + (j * 16)])
                        rows_b[sl] = (rows_b[sl] + rel) * wb

                s_start(rows_b, so_b, sw_b, sem_sb)

            @pl.when(c + 2 * NW < NCHUNKS)
            def _():
                idx_start(c + 2 * NW, ib_b, wv_b, sem_ib)

        return carry

    lax.fori_loop(0, PAIRS, pair, 0)

    for db in range(2):
        rows_b, ib_b, wv_b, so_b, sw_b, sem_gb, sem_ib, sem_sb = bufs[db]
        s_wait(rows_b, so_b, sw_b, sem_sb)
    plsc.subcore_barrier()

    pltpu.sync_copy(acc_sh.at[pl.ds(sid * RPS, RPS)],
                    acc_out.at[cid, 0, pl.ds(sid * RPS, RPS)])
    pltpu.sync_copy(deg_sh.at[pl.ds(sid * RPS, RPS)],
                    deg_out.at[cid, 0, pl.ds(sid * RPS, RPS)])


@functools.cache
def _sc_agg():
    mesh = plsc.VectorSubcoreMesh(core_axis_name="c", subcore_axis_name="s")
    return pl.kernel(
        _sc_body,
        out_type=(
            jax.ShapeDtypeStruct((NC, 1, N_PAD, D), jnp.float32),
            jax.ShapeDtypeStruct((NC, 1, N_PAD), jnp.float32),
        ),
        mesh=mesh,
        compiler_params=pltpu.CompilerParams(needs_layout_passes=False),
        scratch_types=[
            pltpu.VMEM_SHARED((N_PAD, D), jnp.float32),
            pltpu.VMEM_SHARED((N_PAD,), jnp.float32),
            pltpu.VMEM((CH, D), jnp.float32),
            pltpu.VMEM((CH, D), jnp.float32),
            pltpu.VMEM((3, CH), jnp.int32),
            pltpu.VMEM((3, CH), jnp.int32),
            pltpu.VMEM((1, CH), jnp.float32),
            pltpu.VMEM((1, CH), jnp.float32),
            pltpu.VMEM((1, CH), jnp.int32),
            pltpu.VMEM((1, CH), jnp.int32),
            pltpu.VMEM((1, CH), jnp.float32),
            pltpu.VMEM((1, CH), jnp.float32),
            pltpu.VMEM((N_REL, D), jnp.float32),
            pltpu.SemaphoreType.DMA,
            pltpu.SemaphoreType.DMA,
            pltpu.SemaphoreType.DMA,
            pltpu.SemaphoreType.DMA,
            pltpu.SemaphoreType.DMA,
            pltpu.SemaphoreType.DMA,
        ],
    )


def _fin_body(acc_ref, deg_ref, b_ref, o_ref):
    acc = acc_ref[0] + acc_ref[1]
    deg = deg_ref[0] + deg_ref[1]
    num = acc + deg * b_ref[...]
    o_ref[...] = jnp.maximum(num / (deg + 1e-6), 0.0)


def _finalize(acc_p, deg_p, b2):
    blk = 2000
    return pl.pallas_call(
        _fin_body,
        grid=(N_NODES // blk,),
        in_specs=[
            pl.BlockSpec((NC, blk, D), lambda i: (0, i, 0)),
            pl.BlockSpec((NC, blk, 1), lambda i: (0, i, 0)),
            pl.BlockSpec((1, D), lambda i: (0, 0)),
        ],
        out_specs=pl.BlockSpec((blk, D), lambda i: (i, 0)),
        out_shape=jax.ShapeDtypeStruct((N_NODES, D), jnp.float32),
    )(acc_p, deg_p, b2)


def kernel(x, node_in, node_out, relation, edge_weight, W, b, rel_table):
    y = _node_linear(x, W.T)
    nin = node_in.astype(jnp.int32).reshape(NCHUNKS, CH)
    nout = node_out.astype(jnp.int32).reshape(NCHUNKS, CH)
    rel = relation.astype(jnp.int32).reshape(NCHUNKS, CH)
    ed = jnp.stack([nin, nout, rel], axis=1)
    ew = edge_weight.reshape(NCHUNKS, 1, CH)
    za = jnp.zeros((RPS, D), jnp.float32)
    zd = jnp.zeros((RPS,), jnp.float32)
    acc_p, deg_p = _sc_agg()(y, ed, ew, rel_table, za, zd)
    acc_p = acc_p.reshape(NC, N_PAD, D)[:, :N_NODES]
    deg_p = deg_p.reshape(NC, N_PAD, 1)[:, :N_NODES]
    return _finalize(acc_p, deg_p, b.reshape(1, D))
